# baseline (speedup 1.0000x reference)
import functools

import jax
import jax.numpy as jnp
from jax import lax
from jax.experimental import pallas as pl
from jax.experimental.pallas import tpu as pltpu
from jax.experimental.pallas import tpu_sc as plsc

VOCAB = 1000000
BATCH = 4096
SEQ = 200
EMBED_DIM = 64
HIDDEN = 256
NUM_CLASSES = 50

NC = 2
NS = 16
NW = NC * NS
BPW = BATCH // NW
CHUNK_A = 104
CHUNK_B = SEQ - CHUNK_A
LANES = 16
NCOL = EMBED_DIM // LANES
PADDED_DIM = 128


def _pool_body(x_hbm, emb_hbm, out_hbm, idx_v, rows_v, pooled_v, sem0, sem1):
    wid = lax.axis_index("s") * NC + lax.axis_index("c")
    base = wid * BPW
    pltpu.sync_copy(x_hbm.at[pl.ds(base * SEQ, BPW * SEQ)], idx_v)

    sems = (sem0, sem1)

    def issue(b, slot):
        sem = sems[slot]
        pltpu.async_copy(
            emb_hbm.at[idx_v.at[pl.ds(b * SEQ, CHUNK_A)]],
            rows_v.at[pl.ds(slot * SEQ, CHUNK_A)],
            sem,
        )
        pltpu.async_copy(
            emb_hbm.at[idx_v.at[pl.ds(b * SEQ + CHUNK_A, CHUNK_B)]],
            rows_v.at[pl.ds(slot * SEQ + CHUNK_A, CHUNK_B)],
            sem,
        )

    def drain(slot):
        pltpu.make_async_copy(
            emb_hbm.at[pl.ds(0, SEQ)],
            rows_v.at[pl.ds(slot * SEQ, SEQ)],
            sems[slot],
        ).wait()

    inv = jnp.float32(1.0 / SEQ)

    def reduce_item(b, slot):
        def row_step(r, acc):
            return tuple(
                acc[c] + rows_v[slot * SEQ + r, pl.ds(c * LANES, LANES)]
                for c in range(NCOL)
            )

        acc0 = tuple(jnp.zeros((LANES,), jnp.float32) for _ in range(NCOL))
        acc = lax.fori_loop(0, SEQ, row_step, acc0, unroll=4)
        for c in range(NCOL):
            pooled_v[pl.ds(b * EMBED_DIM + c * LANES, LANES)] = acc[c] * inv

    issue(0, 0)
    issue(1, 1)

    def outer(i, _):
        g = i * 2
        for s in range(2):
            b = g + s
            drain(s)
            reduce_item(b, s)
            issue(b + 2, s)
        return 0

    lax.fori_loop(0, (BPW - 2) // 2, outer, 0)
    for s in range(2):
        drain(s)
        reduce_item(BPW - 2 + s, s)

    pltpu.sync_copy(
        pooled_v, out_hbm.at[pl.ds(base * EMBED_DIM, BPW * EMBED_DIM)]
    )


_pool = functools.partial(
    pl.kernel,
    out_type=jax.ShapeDtypeStruct((BATCH * EMBED_DIM,), jnp.float32),
    mesh=plsc.VectorSubcoreMesh(core_axis_name="c", subcore_axis_name="s"),
    compiler_params=pltpu.CompilerParams(use_tc_tiling_on_sc=False),
    scratch_types=[
        pltpu.VMEM((BPW * SEQ,), jnp.int32),
        pltpu.VMEM((2 * SEQ, EMBED_DIM), jnp.float32),
        pltpu.VMEM((BPW * EMBED_DIM,), jnp.float32),
        pltpu.SemaphoreType.DMA,
        pltpu.SemaphoreType.DMA,
    ],
)(_pool_body)


_TR_CHUNK = 128
_TR_NCHUNK = VOCAB // _TR_CHUNK
_TR_TAIL = VOCAB - _TR_NCHUNK * _TR_CHUNK
_TR_BASE = _TR_NCHUNK // NW
_TR_EXTRA = _TR_NCHUNK % NW


def _trans_body(embt_hbm, tail_hbm, out_hbm, in_v, in_t, out_v,
                sem_i0, sem_i1, sem_o0, sem_o1):
    wid = lax.axis_index("s") * NC + lax.axis_index("c")
    chunk0 = wid * _TR_BASE
    sems_i = (sem_i0, sem_i1)
    sems_o = (sem_o0, sem_o1)
    ob = _TR_CHUNK * EMBED_DIM

    def issue_in(ci, slot):
        pltpu.async_copy(
            embt_hbm.at[:, pl.ds((chunk0 + ci) * _TR_CHUNK, _TR_CHUNK)],
            in_v.at[:, pl.ds(slot * _TR_CHUNK, _TR_CHUNK)],
            sems_i[slot],
        )

    def wait_in(slot):
        pltpu.make_async_copy(
            embt_hbm.at[:, pl.ds(0, _TR_CHUNK)],
            in_v.at[:, pl.ds(slot * _TR_CHUNK, _TR_CHUNK)],
            sems_i[slot],
        ).wait()

    lane64 = lax.iota(jnp.int32, LANES) * EMBED_DIM

    def repack(slot):
        def c_step(c, _):
            for t in range(_TR_CHUNK // LANES):
                v = in_v[c, pl.ds(slot * _TR_CHUNK + t * LANES, LANES)]
                idx = lane64 + (t * LANES * EMBED_DIM + c)
                plsc.store_scatter(
                    out_v.at[pl.ds(slot * ob, ob)], [idx], v
                )
            return 0

        lax.fori_loop(0, EMBED_DIM, c_step, 0, unroll=2)

    def issue_out(ci, slot):
        pltpu.async_copy(
            out_v.at[pl.ds(slot * ob, ob)],
            out_hbm.at[pl.ds((chunk0 + ci) * ob, ob)],
            sems_o[slot],
        )

    def wait_out(slot):
        pltpu.make_async_copy(
            out_v.at[pl.ds(slot * ob, ob)],
            out_hbm.at[pl.ds(0, ob)],
            sems_o[slot],
        ).wait()

    issue_in(0, 0)
    issue_in(1, 1)

    def outer(i, _):
        ci = i * 2
        for s in range(2):
            c = ci + s
            wait_in(s)

            @pl.when(c >= 2)
            def _():
                wait_out(s)

            repack(s)
            issue_out(c, s)

            @pl.when(c + 2 < _TR_BASE)
            def _():
                issue_in(c + 2, s)

        return 0

    lax.fori_loop(0, _TR_BASE // 2, outer, 0)
    wait_out(0)
    wait_out(1)

    @pl.when(wid < _TR_EXTRA)
    def _():
        ce = NW * _TR_BASE + wid
        pltpu.sync_copy(
            embt_hbm.at[:, pl.ds(ce * _TR_CHUNK, _TR_CHUNK)],
            in_v.at[:, pl.ds(0, _TR_CHUNK)],
        )
        repack(0)
        pltpu.sync_copy(
            out_v.at[pl.ds(0, ob)],
            out_hbm.at[pl.ds(ce * ob, ob)],
        )

    @pl.when(wid == _TR_EXTRA)
    def _():
        pltpu.sync_copy(tail_hbm, in_t)

        def c_step(c, _):
            for t in range(_TR_TAIL // LANES):
                v = in_t[c, pl.ds(t * LANES, LANES)]
                idx = lane64 + (t * LANES * EMBED_DIM + c)
                plsc.store_scatter(
                    out_v.at[pl.ds(0, _TR_TAIL * EMBED_DIM)], [idx], v
                )
            return 0

        lax.fori_loop(0, EMBED_DIM, c_step, 0)
        pltpu.sync_copy(
            out_v.at[pl.ds(0, _TR_TAIL * EMBED_DIM)],
            out_hbm.at[pl.ds(_TR_NCHUNK * _TR_CHUNK * EMBED_DIM,
                             _TR_TAIL * EMBED_DIM)],
        )


_trans = functools.partial(
    pl.kernel,
    out_type=jax.ShapeDtypeStruct((VOCAB * EMBED_DIM,), jnp.float32),
    mesh=plsc.VectorSubcoreMesh(core_axis_name="c", subcore_axis_name="s"),
    compiler_params=pltpu.CompilerParams(needs_layout_passes=False),
    scratch_types=[
        pltpu.VMEM((EMBED_DIM, 2 * _TR_CHUNK), jnp.float32),
        pltpu.VMEM((EMBED_DIM, _TR_TAIL), jnp.float32),
        pltpu.VMEM((2 * _TR_CHUNK * EMBED_DIM,), jnp.float32),
        pltpu.SemaphoreType.DMA,
        pltpu.SemaphoreType.DMA,
        pltpu.SemaphoreType.DMA,
        pltpu.SemaphoreType.DMA,
    ],
)(_trans_body)


def _mlp_body(p_ref, w1_ref, b1_ref, w2_ref, b2_ref, o_ref):
    h = jnp.dot(p_ref[...], w1_ref[...], preferred_element_type=jnp.float32)
    h = jnp.maximum(h + b1_ref[...], 0.0)
    o_ref[...] = (
        jnp.dot(h, w2_ref[...], preferred_element_type=jnp.float32)
        + b2_ref[...]
    )


_MLP_BLOCK = 512


@jax.jit
def kernel(x, emb, W1, b1, W2, b2):
    x = x.astype(jnp.int32).reshape(BATCH * SEQ)
    tail = emb.T[:, _TR_NCHUNK * _TR_CHUNK:]
    emb_lin = _trans(emb.T, tail).reshape(VOCAB, EMBED_DIM)
    pooled = _pool(x, emb_lin).reshape(BATCH, EMBED_DIM)
    grid = BATCH // _MLP_BLOCK
    out = pl.pallas_call(
        _mlp_body,
        grid=(grid,),
        in_specs=[
            pl.BlockSpec((_MLP_BLOCK, EMBED_DIM), lambda i: (i, 0)),
            pl.BlockSpec((EMBED_DIM, HIDDEN), lambda i: (0, 0)),
            pl.BlockSpec((1, HIDDEN), lambda i: (0, 0)),
            pl.BlockSpec((HIDDEN, NUM_CLASSES), lambda i: (0, 0)),
            pl.BlockSpec((1, NUM_CLASSES), lambda i: (0, 0)),
        ],
        out_specs=pl.BlockSpec((_MLP_BLOCK, NUM_CLASSES), lambda i: (i, 0)),
        out_shape=jax.ShapeDtypeStruct((BATCH, NUM_CLASSES), jnp.float32),
    )(pooled, W1, b1.reshape(1, HIDDEN), W2, b2.reshape(1, NUM_CLASSES))
    return out


# speedup vs baseline: 2.1273x; 2.1273x over previous
import functools

import jax
import jax.numpy as jnp
from jax import lax
from jax.experimental import pallas as pl
from jax.experimental.pallas import tpu as pltpu
from jax.experimental.pallas import tpu_sc as plsc

VOCAB = 1000000
BATCH = 4096
SEQ = 200
EMBED_DIM = 64
HIDDEN = 256
NUM_CLASSES = 50

NC = 2
NS = 16
NW = NC * NS
BPW = BATCH // NW
CHUNK_A = 104
CHUNK_B = SEQ - CHUNK_A
LANES = 16
NCOL = EMBED_DIM // LANES
PADDED_DIM = 128


def _pool_body(x_hbm, emb_hbm, out_hbm, idx_v, rows_v, pooled_v, sem0, sem1):
    wid = lax.axis_index("s") * NC + lax.axis_index("c")
    base = wid * BPW
    pltpu.sync_copy(x_hbm.at[pl.ds(base * SEQ, BPW * SEQ)], idx_v)

    sems = (sem0, sem1)

    def issue(b, slot):
        sem = sems[slot]
        pltpu.async_copy(
            emb_hbm.at[idx_v.at[pl.ds(b * SEQ, CHUNK_A)]],
            rows_v.at[pl.ds(slot * SEQ, CHUNK_A)],
            sem,
        )
        pltpu.async_copy(
            emb_hbm.at[idx_v.at[pl.ds(b * SEQ + CHUNK_A, CHUNK_B)]],
            rows_v.at[pl.ds(slot * SEQ + CHUNK_A, CHUNK_B)],
            sem,
        )

    def drain(slot):
        pltpu.make_async_copy(
            emb_hbm.at[pl.ds(0, SEQ)],
            rows_v.at[pl.ds(slot * SEQ, SEQ)],
            sems[slot],
        ).wait()

    inv = jnp.float32(1.0 / SEQ)

    def reduce_item(b, slot):
        def row_step(r, acc):
            return tuple(
                acc[c] + rows_v[slot * SEQ + r, pl.ds(c * LANES, LANES)]
                for c in range(NCOL)
            )

        acc0 = tuple(jnp.zeros((LANES,), jnp.float32) for _ in range(NCOL))
        acc = lax.fori_loop(0, SEQ, row_step, acc0, unroll=4)
        for c in range(NCOL):
            pooled_v[pl.ds(b * EMBED_DIM + c * LANES, LANES)] = acc[c] * inv

    issue(0, 0)
    issue(1, 1)

    def outer(i, _):
        g = i * 2
        for s in range(2):
            b = g + s
            drain(s)
            reduce_item(b, s)
            issue(b + 2, s)
        return 0

    lax.fori_loop(0, (BPW - 2) // 2, outer, 0)
    for s in range(2):
        drain(s)
        reduce_item(BPW - 2 + s, s)

    pltpu.sync_copy(
        pooled_v, out_hbm.at[pl.ds(base * EMBED_DIM, BPW * EMBED_DIM)]
    )


_pool = functools.partial(
    pl.kernel,
    out_type=jax.ShapeDtypeStruct((BATCH * EMBED_DIM,), jnp.float32),
    mesh=plsc.VectorSubcoreMesh(core_axis_name="c", subcore_axis_name="s"),
    compiler_params=pltpu.CompilerParams(use_tc_tiling_on_sc=False),
    scratch_types=[
        pltpu.VMEM((BPW * SEQ,), jnp.int32),
        pltpu.VMEM((2 * SEQ, EMBED_DIM), jnp.float32),
        pltpu.VMEM((BPW * EMBED_DIM,), jnp.float32),
        pltpu.SemaphoreType.DMA,
        pltpu.SemaphoreType.DMA,
    ],
)(_pool_body)


_TR_CHUNK = 128
_TR_NCHUNK = VOCAB // _TR_CHUNK
_TR_TAIL = VOCAB - _TR_NCHUNK * _TR_CHUNK
_TR_BASE = _TR_NCHUNK // NW
_TR_EXTRA = _TR_NCHUNK % NW


def _trans_body(embt_hbm, tail_hbm, out_hbm, in_a, in_b, in_t, out_v,
                sem_i0, sem_i1, sem_o0, sem_o1):
    wid = lax.axis_index("s") * NC + lax.axis_index("c")
    chunk0 = wid * _TR_BASE
    sems_i = (sem_i0, sem_i1)
    sems_o = (sem_o0, sem_o1)
    in_bufs = (in_a, in_b)
    ob = _TR_CHUNK * EMBED_DIM

    def issue_in(ci, slot):
        pltpu.async_copy(
            embt_hbm.at[:, pl.ds((chunk0 + ci) * _TR_CHUNK, _TR_CHUNK)],
            in_bufs[slot],
            sems_i[slot],
        )

    def wait_in(slot):
        pltpu.make_async_copy(
            embt_hbm.at[:, pl.ds(0, _TR_CHUNK)],
            in_bufs[slot],
            sems_i[slot],
        ).wait()

    iota16 = lax.iota(jnp.int32, LANES)
    rot = [lax.rem(iota16 + d, LANES) for d in range(LANES)]
    rot64 = [rot[d] * EMBED_DIM + iota16 for d in range(LANES)]

    def repack(slot):
        buf = in_bufs[slot]
        for jc in range(EMBED_DIM // LANES):
            c_idx = iota16 + jc * LANES

            def r_step(rb, _):
                r0 = rb * LANES
                obase = slot * ob + r0 * EMBED_DIM + jc * LANES
                for d in range(LANES):
                    v = plsc.load_gather(buf, [c_idx, rot[d] + r0])
                    plsc.store_scatter(out_v, [rot64[d] + obase], v)
                return 0

            lax.fori_loop(0, _TR_CHUNK // LANES, r_step, 0)

    def issue_out(ci, slot):
        pltpu.async_copy(
            out_v.at[pl.ds(slot * ob, ob)],
            out_hbm.at[pl.ds((chunk0 + ci) * ob, ob)],
            sems_o[slot],
        )

    def wait_out(slot):
        pltpu.make_async_copy(
            out_v.at[pl.ds(slot * ob, ob)],
            out_hbm.at[pl.ds(0, ob)],
            sems_o[slot],
        ).wait()

    issue_in(0, 0)
    issue_in(1, 1)

    def outer(i, _):
        ci = i * 2
        for s in range(2):
            c = ci + s
            wait_in(s)

            @pl.when(c >= 2)
            def _():
                wait_out(s)

            repack(s)
            issue_out(c, s)

            @pl.when(c + 2 < _TR_BASE)
            def _():
                issue_in(c + 2, s)

        return 0

    lax.fori_loop(0, _TR_BASE // 2, outer, 0)
    wait_out(0)
    wait_out(1)

    @pl.when(wid < _TR_EXTRA)
    def _():
        ce = NW * _TR_BASE + wid
        pltpu.sync_copy(
            embt_hbm.at[:, pl.ds(ce * _TR_CHUNK, _TR_CHUNK)],
            in_bufs[0],
        )
        repack(0)
        pltpu.sync_copy(
            out_v.at[pl.ds(0, ob)],
            out_hbm.at[pl.ds(ce * ob, ob)],
        )

    @pl.when(wid == _TR_EXTRA)
    def _():
        pltpu.sync_copy(tail_hbm, in_t)
        lane64 = iota16 * EMBED_DIM

        def c_step(c, _):
            for t in range(_TR_TAIL // LANES):
                v = in_t[c, pl.ds(t * LANES, LANES)]
                idx = lane64 + (t * LANES * EMBED_DIM + c)
                plsc.store_scatter(
                    out_v.at[pl.ds(0, _TR_TAIL * EMBED_DIM)], [idx], v
                )
            return 0

        lax.fori_loop(0, EMBED_DIM, c_step, 0)
        pltpu.sync_copy(
            out_v.at[pl.ds(0, _TR_TAIL * EMBED_DIM)],
            out_hbm.at[pl.ds(_TR_NCHUNK * _TR_CHUNK * EMBED_DIM,
                             _TR_TAIL * EMBED_DIM)],
        )


_trans = functools.partial(
    pl.kernel,
    out_type=jax.ShapeDtypeStruct((VOCAB * EMBED_DIM,), jnp.float32),
    mesh=plsc.VectorSubcoreMesh(core_axis_name="c", subcore_axis_name="s"),
    compiler_params=pltpu.CompilerParams(needs_layout_passes=False),
    scratch_types=[
        pltpu.VMEM((EMBED_DIM, _TR_CHUNK), jnp.float32),
        pltpu.VMEM((EMBED_DIM, _TR_CHUNK), jnp.float32),
        pltpu.VMEM((EMBED_DIM, _TR_TAIL), jnp.float32),
        pltpu.VMEM((2 * _TR_CHUNK * EMBED_DIM,), jnp.float32),
        pltpu.SemaphoreType.DMA,
        pltpu.SemaphoreType.DMA,
        pltpu.SemaphoreType.DMA,
        pltpu.SemaphoreType.DMA,
    ],
)(_trans_body)


def _mlp_body(p_ref, w1_ref, b1_ref, w2_ref, b2_ref, o_ref):
    h = jnp.dot(p_ref[...], w1_ref[...], preferred_element_type=jnp.float32)
    h = jnp.maximum(h + b1_ref[...], 0.0)
    o_ref[...] = (
        jnp.dot(h, w2_ref[...], preferred_element_type=jnp.float32)
        + b2_ref[...]
    )


_MLP_BLOCK = 512


@jax.jit
def kernel(x, emb, W1, b1, W2, b2):
    x = x.astype(jnp.int32).reshape(BATCH * SEQ)
    tail = emb.T[:, _TR_NCHUNK * _TR_CHUNK:]
    emb_lin = _trans(emb.T, tail).reshape(VOCAB, EMBED_DIM)
    pooled = _pool(x, emb_lin).reshape(BATCH, EMBED_DIM)
    grid = BATCH // _MLP_BLOCK
    out = pl.pallas_call(
        _mlp_body,
        grid=(grid,),
        in_specs=[
            pl.BlockSpec((_MLP_BLOCK, EMBED_DIM), lambda i: (i, 0)),
            pl.BlockSpec((EMBED_DIM, HIDDEN), lambda i: (0, 0)),
            pl.BlockSpec((1, HIDDEN), lambda i: (0, 0)),
            pl.BlockSpec((HIDDEN, NUM_CLASSES), lambda i: (0, 0)),
            pl.BlockSpec((1, NUM_CLASSES), lambda i: (0, 0)),
        ],
        out_specs=pl.BlockSpec((_MLP_BLOCK, NUM_CLASSES), lambda i: (i, 0)),
        out_shape=jax.ShapeDtypeStruct((BATCH, NUM_CLASSES), jnp.float32),
    )(pooled, W1, b1.reshape(1, HIDDEN), W2, b2.reshape(1, NUM_CLASSES))
    return out
